# baseline (speedup 1.0000x reference)
import functools

import jax
import jax.numpy as jnp
from jax import lax
from jax.experimental import pallas as pl
from jax.experimental.pallas import tpu as pltpu
from jax.experimental.pallas import tpu_sc as plsc

N = 10000
E = 160000
CONV = 128
NG = 64
HID = 512

BN = 1000
NSTEP = N // BN
BE = 4000

K = 40
NW = 32
NCHUNK = E // K
BASE_CHUNKS = NCHUNK // NW
T0 = 624
TAIL = N - 16 * T0
TAIL_OFF = 16 * T0


def _lr(v):
    return jnp.where(v > 0, v, 0.01 * v)


def _prep_body(x_ref, wi_ref, bi_ref, wd_ref, ws_ref, h_ref, a_ref, b_ref):
    h = _lr(jnp.dot(x_ref[...], wi_ref[...], preferred_element_type=jnp.float32)
            + bi_ref[...])
    h_ref[...] = h
    a_ref[...] = jnp.dot(h, wd_ref[...], preferred_element_type=jnp.float32)
    b_ref[...] = jnp.dot(h, ws_ref[...], preferred_element_type=jnp.float32)


_prep = pl.pallas_call(
    _prep_body,
    grid=(NSTEP,),
    in_specs=[
        pl.BlockSpec((BN, 128), lambda i: (i, 0)),
        pl.BlockSpec((128, 128), lambda i: (0, 0)),
        pl.BlockSpec((1, 128), lambda i: (0, 0)),
        pl.BlockSpec((128, 256), lambda i: (0, 0)),
        pl.BlockSpec((128, 256), lambda i: (0, 0)),
    ],
    out_specs=[
        pl.BlockSpec((BN, 128), lambda i: (i, 0)),
        pl.BlockSpec((BN, 256), lambda i: (i, 0)),
        pl.BlockSpec((BN, 256), lambda i: (i, 0)),
    ],
    out_shape=[
        jax.ShapeDtypeStruct((N, 128), jnp.float32),
        jax.ShapeDtypeStruct((N, 256), jnp.float32),
        jax.ShapeDtypeStruct((N, 256), jnp.float32),
    ],
)


def _edgeattr_body(e_ref, w_ref, b_ref, ea0_ref, ea1_ref):
    z = jnp.dot(e_ref[...], w_ref[...], preferred_element_type=jnp.float32) + b_ref[...]
    ea0_ref[...] = z[:, :256]
    ea1_ref[...] = z[:, 256:]


_edgeattr = pl.pallas_call(
    _edgeattr_body,
    grid=(E // BE,),
    in_specs=[
        pl.BlockSpec((BE, 16), lambda i: (i, 0)),
        pl.BlockSpec((16, 512), lambda i: (0, 0)),
        pl.BlockSpec((1, 512), lambda i: (0, 0)),
    ],
    out_specs=[
        pl.BlockSpec((BE, 256), lambda i: (i, 0)),
        pl.BlockSpec((BE, 256), lambda i: (i, 0)),
    ],
    out_shape=[
        jax.ShapeDtypeStruct((E, 256), jnp.float32),
        jax.ShapeDtypeStruct((E, 256), jnp.float32),
    ],
)


def _sc_edge_body(a_hbm, b_hbm, ea_hbm, dst_hbm, src_hbm, z_hbm, out_hbm,
                  dvi, svi, abuf, bbuf, ebuf, mbuf, aggr, sema, semb, seme):
    cid = lax.axis_index("c")
    sid = lax.axis_index("s")
    wid = cid * 16 + sid

    pltpu.sync_copy(z_hbm, aggr.at[pl.ds(sid * T0, T0)])

    @pl.when(sid == 15)
    def _():
        pltpu.sync_copy(z_hbm.at[pl.ds(0, TAIL)], aggr.at[pl.ds(TAIL_OFF, TAIL)])

    plsc.subcore_barrier()

    def do_chunk(c):
        base = c * K
        pltpu.sync_copy(dst_hbm.at[pl.ds(base, K)], dvi)
        pltpu.sync_copy(src_hbm.at[pl.ds(base, K)], svi)
        cpa = pltpu.async_copy(a_hbm.at[dvi], abuf, sema)
        cpb = pltpu.async_copy(b_hbm.at[svi], bbuf, semb)
        cpe = pltpu.async_copy(ea_hbm.at[pl.ds(base, K)], ebuf, seme)
        cpa.wait()
        cpb.wait()
        cpe.wait()

        def row(r, carry):
            for j in range(8):
                lo = j * 16
                gf = (abuf[r, pl.ds(lo, 16)] + bbuf[r, pl.ds(lo, 16)]
                      + ebuf[r, pl.ds(lo, 16)])
                gs = (abuf[r, pl.ds(128 + lo, 16)] + bbuf[r, pl.ds(128 + lo, 16)]
                      + ebuf[r, pl.ds(128 + lo, 16)])
                sig = 1.0 / (1.0 + jnp.exp(-gf))
                u = jnp.exp(-jnp.abs(gs))
                t = u / (2.0 + u)
                t2 = t * t
                p = t2 * (1.0 / 7.0) + (1.0 / 5.0)
                p = t2 * p + (1.0 / 3.0)
                p = t2 * p + 1.0
                sp = jnp.maximum(gs, 0.0) + (2.0 * t) * p
                mbuf[r, pl.ds(lo, 16)] = sig * sp
            return carry

        lax.fori_loop(0, K, row, 0)
        pltpu.sync_copy(mbuf, aggr.at[dvi], add=True)

    @pl.loop(0, BASE_CHUNKS)
    def _(i):
        do_chunk(wid + i * NW)

    plsc.subcore_barrier()
    pltpu.sync_copy(aggr.at[pl.ds(sid * T0, T0)],
                    out_hbm.at[cid, pl.ds(sid * T0, T0)])

    @pl.when(sid == 15)
    def _():
        pltpu.sync_copy(aggr.at[pl.ds(TAIL_OFF, TAIL)],
                        out_hbm.at[cid, pl.ds(TAIL_OFF, TAIL)])


_edge_stage = functools.partial(
    pl.kernel,
    out_type=jax.ShapeDtypeStruct((2, N, CONV), jnp.float32),
    mesh=plsc.VectorSubcoreMesh(core_axis_name="c", subcore_axis_name="s"),
    scratch_types=[
        pltpu.VMEM((K,), jnp.int32),
        pltpu.VMEM((K,), jnp.int32),
        pltpu.VMEM((K, 256), jnp.float32),
        pltpu.VMEM((K, 256), jnp.float32),
        pltpu.VMEM((K, 256), jnp.float32),
        pltpu.VMEM((K, CONV), jnp.float32),
        pltpu.VMEM_SHARED((N, CONV), jnp.float32),
        pltpu.SemaphoreType.DMA,
        pltpu.SemaphoreType.DMA,
        pltpu.SemaphoreType.DMA,
    ],
)(_sc_edge_body)


def _stats_body(part_ref, stats_ref):
    i = pl.program_id(0)
    aggr = part_ref[0] + part_ref[1]
    s = jnp.sum(aggr, axis=0)
    s2 = jnp.sum(aggr * aggr, axis=0)
    st = jnp.stack([s, s2])

    @pl.when(i == 0)
    def _():
        stats_ref[...] = st

    @pl.when(i > 0)
    def _():
        stats_ref[...] += st


_stats = pl.pallas_call(
    _stats_body,
    grid=(NSTEP,),
    in_specs=[pl.BlockSpec((2, BN, 128), lambda i: (0, i, 0))],
    out_specs=pl.BlockSpec((2, 128), lambda i: (0, 0)),
    out_shape=jax.ShapeDtypeStruct((2, 128), jnp.float32),
)


def _apply0_body(part_ref, h_ref, stats_ref, g_ref, be_ref, wl_ref, bl_ref,
                 wd_ref, ws_ref, h1_ref, a_ref, b_ref):
    aggr = part_ref[0] + part_ref[1]
    mean = stats_ref[0:1, :] / N
    var = stats_ref[1:2, :] / N - mean * mean
    inv = lax.rsqrt(var + 1e-5)
    xn = (aggr - mean) * (inv * g_ref[...]) + be_ref[...] + h_ref[...]
    hn = _lr(jnp.dot(xn, wl_ref[...], preferred_element_type=jnp.float32)
             + bl_ref[...])
    h1_ref[...] = hn
    a_ref[...] = jnp.dot(hn, wd_ref[...], preferred_element_type=jnp.float32)
    b_ref[...] = jnp.dot(hn, ws_ref[...], preferred_element_type=jnp.float32)


_apply0 = pl.pallas_call(
    _apply0_body,
    grid=(NSTEP,),
    in_specs=[
        pl.BlockSpec((2, BN, 128), lambda i: (0, i, 0)),
        pl.BlockSpec((BN, 128), lambda i: (i, 0)),
        pl.BlockSpec((2, 128), lambda i: (0, 0)),
        pl.BlockSpec((1, 128), lambda i: (0, 0)),
        pl.BlockSpec((1, 128), lambda i: (0, 0)),
        pl.BlockSpec((128, 128), lambda i: (0, 0)),
        pl.BlockSpec((1, 128), lambda i: (0, 0)),
        pl.BlockSpec((128, 256), lambda i: (0, 0)),
        pl.BlockSpec((128, 256), lambda i: (0, 0)),
    ],
    out_specs=[
        pl.BlockSpec((BN, 128), lambda i: (i, 0)),
        pl.BlockSpec((BN, 256), lambda i: (i, 0)),
        pl.BlockSpec((BN, 256), lambda i: (i, 0)),
    ],
    out_shape=[
        jax.ShapeDtypeStruct((N, 128), jnp.float32),
        jax.ShapeDtypeStruct((N, 256), jnp.float32),
        jax.ShapeDtypeStruct((N, 256), jnp.float32),
    ],
)


def _final_body(part_ref, h_ref, stats_ref, g_ref, be_ref, wl_ref, bl_ref,
                batch_ref, wp_ref, bp_ref, wh_ref, bh_ref, wo_ref, bo_ref,
                out_ref, pooled_ref, cnt_ref):
    i = pl.program_id(0)
    aggr = part_ref[0] + part_ref[1]
    mean = stats_ref[0:1, :] / N
    var = stats_ref[1:2, :] / N - mean * mean
    inv = lax.rsqrt(var + 1e-5)
    xn = (aggr - mean) * (inv * g_ref[...]) + be_ref[...] + h_ref[...]
    h2 = _lr(jnp.dot(xn, wl_ref[...], preferred_element_type=jnp.float32)
             + bl_ref[...])

    batch_blk = batch_ref[0, 0, :]
    gids = lax.broadcasted_iota(jnp.int32, (BN, NG), 1)
    onehot = (batch_blk[:, None] == gids).astype(jnp.float32)
    pooled_c = lax.dot_general(onehot, h2, (((0,), (0,)), ((), ())),
                               preferred_element_type=jnp.float32)
    cnt_c = jnp.broadcast_to(jnp.sum(onehot, axis=0)[:, None], (NG, 128))

    @pl.when(i == 0)
    def _():
        pooled_ref[...] = pooled_c
        cnt_ref[...] = cnt_c

    @pl.when(i > 0)
    def _():
        pooled_ref[...] += pooled_c
        cnt_ref[...] += cnt_c

    @pl.when(i == NSTEP - 1)
    def _():
        pooled = pooled_ref[...] / jnp.maximum(cnt_ref[...], 1.0)
        hid = _lr(jnp.dot(pooled, wp_ref[...], preferred_element_type=jnp.float32)
                  + bp_ref[...])
        hid = _lr(jnp.dot(hid, wh_ref[...], preferred_element_type=jnp.float32)
                  + bh_ref[...])
        out_ref[...] = _lr(jnp.dot(hid, wo_ref[...],
                                   preferred_element_type=jnp.float32)
                           + bo_ref[...])


_final = pl.pallas_call(
    _final_body,
    grid=(NSTEP,),
    in_specs=[
        pl.BlockSpec((2, BN, 128), lambda i: (0, i, 0)),
        pl.BlockSpec((BN, 128), lambda i: (i, 0)),
        pl.BlockSpec((2, 128), lambda i: (0, 0)),
        pl.BlockSpec((1, 128), lambda i: (0, 0)),
        pl.BlockSpec((1, 128), lambda i: (0, 0)),
        pl.BlockSpec((128, 128), lambda i: (0, 0)),
        pl.BlockSpec((1, 128), lambda i: (0, 0)),
        pl.BlockSpec((1, 1, BN), lambda i: (i, 0, 0)),
        pl.BlockSpec((128, HID), lambda i: (0, 0)),
        pl.BlockSpec((1, HID), lambda i: (0, 0)),
        pl.BlockSpec((HID, HID), lambda i: (0, 0)),
        pl.BlockSpec((1, HID), lambda i: (0, 0)),
        pl.BlockSpec((HID, 1), lambda i: (0, 0)),
        pl.BlockSpec((1, 1), lambda i: (0, 0)),
    ],
    out_specs=pl.BlockSpec((NG, 1), lambda i: (0, 0)),
    out_shape=jax.ShapeDtypeStruct((NG, 1), jnp.float32),
    scratch_shapes=[
        pltpu.VMEM((NG, 128), jnp.float32),
        pltpu.VMEM((NG, 128), jnp.float32),
    ],
)


def kernel(x, edge_index, edge_attr, batch,
           W_init, b_init,
           Wf0, bf0, Ws0, bs0, gamma0, beta0, Wl0, bl0,
           Wf1, bf1, Ws1, bs1, gamma1, beta1, Wl1, bl1,
           W_pool, b_pool, W_h1, b_h1, W_out, b_out):
    src = edge_index[0]
    dst = edge_index[1]
    zrows = jnp.zeros((T0, CONV), jnp.float32)

    Wd0 = jnp.concatenate([Wf0[:CONV], Ws0[:CONV]], axis=1)
    Wsrc0 = jnp.concatenate([Wf0[CONV:2 * CONV], Ws0[CONV:2 * CONV]], axis=1)
    Wd1 = jnp.concatenate([Wf1[:CONV], Ws1[:CONV]], axis=1)
    Wsrc1 = jnp.concatenate([Wf1[CONV:2 * CONV], Ws1[CONV:2 * CONV]], axis=1)
    We = jnp.concatenate([Wf0[2 * CONV:], Ws0[2 * CONV:],
                          Wf1[2 * CONV:], Ws1[2 * CONV:]], axis=1)
    biases = jnp.concatenate([bf0, bs0, bf1, bs1]).reshape(1, 512)

    h0, A0, B0 = _prep(x, W_init, b_init.reshape(1, 128), Wd0, Wsrc0)
    EA0, EA1 = _edgeattr(edge_attr, We, biases)

    part0 = _edge_stage(A0, B0, EA0, dst, src, zrows)
    stats0 = _stats(part0)
    h1, A1, B1 = _apply0(part0, h0, stats0, gamma0.reshape(1, 128),
                         beta0.reshape(1, 128), Wl0, bl0.reshape(1, 128),
                         Wd1, Wsrc1)

    part1 = _edge_stage(A1, B1, EA1, dst, src, zrows)
    stats1 = _stats(part1)

    out = _final(part1, h1, stats1, gamma1.reshape(1, 128),
                 beta1.reshape(1, 128), Wl1, bl1.reshape(1, 128),
                 batch.reshape(NSTEP, 1, BN),
                 W_pool, b_pool.reshape(1, HID),
                 W_h1, b_h1.reshape(1, HID),
                 W_out, b_out.reshape(1, 1))
    return out


# speedup vs baseline: 1.0398x; 1.0398x over previous
import jax
import jax.numpy as jnp
from jax import lax
from jax.experimental import pallas as pl
from jax.experimental.pallas import tpu as pltpu
from jax.experimental.pallas import tpu_sc as plsc

N = 10000
E = 160000
CONV = 128
NG = 64
HID = 512

BN = 1000
NSTEP = N // BN

K = 24
NW = 32
CPW = 210
EPW = K * CPW
E_PAD = EPW * NW
BE = 4032
T0 = 624
TAIL = N - 16 * T0
TAIL_OFF = 16 * T0


def _lr(v):
    return jnp.where(v > 0, v, 0.01 * v)


def _prep_body(x_ref, wi_ref, bi_ref, wd_ref, ws_ref, h_ref, a_ref, b_ref):
    h = _lr(jnp.dot(x_ref[...], wi_ref[...], preferred_element_type=jnp.float32)
            + bi_ref[...])
    h_ref[...] = h
    a_ref[...] = jnp.dot(h, wd_ref[...], preferred_element_type=jnp.float32)
    b_ref[...] = jnp.dot(h, ws_ref[...], preferred_element_type=jnp.float32)


_prep = pl.pallas_call(
    _prep_body,
    grid=(NSTEP,),
    in_specs=[
        pl.BlockSpec((BN, 128), lambda i: (i, 0)),
        pl.BlockSpec((128, 128), lambda i: (0, 0)),
        pl.BlockSpec((1, 128), lambda i: (0, 0)),
        pl.BlockSpec((128, 256), lambda i: (0, 0)),
        pl.BlockSpec((128, 256), lambda i: (0, 0)),
    ],
    out_specs=[
        pl.BlockSpec((BN, 128), lambda i: (i, 0)),
        pl.BlockSpec((BN, 256), lambda i: (i, 0)),
        pl.BlockSpec((BN, 256), lambda i: (i, 0)),
    ],
    out_shape=[
        jax.ShapeDtypeStruct((N, 128), jnp.float32),
        jax.ShapeDtypeStruct((N, 256), jnp.float32),
        jax.ShapeDtypeStruct((N, 256), jnp.float32),
    ],
)


def _edgeattr_body(e_ref, w_ref, b_ref, ea0_ref, ea1_ref):
    i = pl.program_id(0)
    z = jnp.dot(e_ref[...], w_ref[...], preferred_element_type=jnp.float32) + b_ref[...]
    rid = lax.broadcasted_iota(jnp.int32, (BE, 1), 0) + i * BE
    z = jnp.where(rid < E, z, -50.0)
    ea0_ref[...] = z[:, :256]
    ea1_ref[...] = z[:, 256:]


_edgeattr = pl.pallas_call(
    _edgeattr_body,
    grid=(E_PAD // BE,),
    in_specs=[
        pl.BlockSpec((BE, 16), lambda i: (i, 0)),
        pl.BlockSpec((16, 512), lambda i: (0, 0)),
        pl.BlockSpec((1, 512), lambda i: (0, 0)),
    ],
    out_specs=[
        pl.BlockSpec((BE, 256), lambda i: (i, 0)),
        pl.BlockSpec((BE, 256), lambda i: (i, 0)),
    ],
    out_shape=[
        jax.ShapeDtypeStruct((E_PAD, 256), jnp.float32),
        jax.ShapeDtypeStruct((E_PAD, 256), jnp.float32),
    ],
)


def _sc_edge_body(a_hbm, b_hbm, ea_hbm, dst_hbm, src_hbm, z_hbm, out_hbm,
                  dvall, svall, dsc,
                  ab0, bb0, eb0, ab1, bb1, eb1, mb,
                  aggr, sa0, sb0, se0, sa1, sb1, se1):
    cid = lax.axis_index("c")
    sid = lax.axis_index("s")
    wid = cid * 16 + sid

    set0 = (ab0, bb0, eb0, sa0, sb0, se0)
    set1 = (ab1, bb1, eb1, sa1, sb1, se1)

    pltpu.sync_copy(z_hbm, aggr.at[pl.ds(sid * T0, T0)])

    @pl.when(sid == 15)
    def _():
        pltpu.sync_copy(z_hbm.at[pl.ds(0, TAIL)], aggr.at[pl.ds(TAIL_OFF, TAIL)])

    pltpu.sync_copy(dst_hbm.at[pl.ds(wid * EPW, EPW)], dvall)
    pltpu.sync_copy(src_hbm.at[pl.ds(wid * EPW, EPW)], svall)
    plsc.subcore_barrier()

    def issue(i, s):
        ab, bb, eb, sa, sb, se = s
        base = wid * EPW + i * K
        da = pltpu.async_copy(a_hbm.at[dvall.at[pl.ds(i * K, K)]], ab, sa)
        db = pltpu.async_copy(b_hbm.at[svall.at[pl.ds(i * K, K)]], bb, sb)
        de = pltpu.async_copy(ea_hbm.at[pl.ds(base, K)], eb, se)
        return (da, db, de)

    def finish(i, s, descs):
        ab, bb, eb = s[0], s[1], s[2]
        for d in descs:
            d.wait()

        def row(r, carry):
            for j in range(8):
                lo = j * 16
                gf = (ab[r, pl.ds(lo, 16)] + bb[r, pl.ds(lo, 16)]
                      + eb[r, pl.ds(lo, 16)])
                gs = (ab[r, pl.ds(128 + lo, 16)] + bb[r, pl.ds(128 + lo, 16)]
                      + eb[r, pl.ds(128 + lo, 16)])
                sig = 1.0 / (1.0 + jnp.exp(-gf))
                u = jnp.exp(-jnp.abs(gs))
                t = u / (2.0 + u)
                t2 = t * t
                q = t2 * (1.0 / 7.0) + (1.0 / 5.0)
                q = t2 * q + (1.0 / 3.0)
                q = t2 * q + 1.0
                sp = jnp.maximum(gs, 0.0) + (2.0 * t) * q
                mb[r, pl.ds(lo, 16)] = sig * sp
            return carry

        lax.fori_loop(0, K, row, 0)
        o = 0
        while o + 16 <= K:
            dsc[pl.ds(o, 16)] = dvall[pl.ds(i * K + o, 16)]
            o += 16
        if o < K:
            dsc[pl.ds(K - 16, 16)] = dvall[pl.ds(i * K + K - 16, 16)]
        pltpu.sync_copy(mb, aggr.at[dsc], add=True)

    @pl.loop(0, CPW // 2)
    def _(t):
        i0 = 2 * t
        i1 = i0 + 1
        d0 = issue(i0, set0)
        d1 = issue(i1, set1)
        finish(i0, set0, d0)
        finish(i1, set1, d1)

    plsc.subcore_barrier()
    pltpu.sync_copy(aggr.at[pl.ds(sid * T0, T0)],
                    out_hbm.at[cid, pl.ds(sid * T0, T0)])

    @pl.when(sid == 15)
    def _():
        pltpu.sync_copy(aggr.at[pl.ds(TAIL_OFF, TAIL)],
                        out_hbm.at[cid, pl.ds(TAIL_OFF, TAIL)])


_edge_stage = pl.kernel(
    _sc_edge_body,
    out_type=jax.ShapeDtypeStruct((2, N, CONV), jnp.float32),
    mesh=plsc.VectorSubcoreMesh(core_axis_name="c", subcore_axis_name="s"),
    scratch_types=[
        pltpu.VMEM((EPW,), jnp.int32),
        pltpu.VMEM((EPW,), jnp.int32),
        pltpu.VMEM((K,), jnp.int32),
        pltpu.VMEM((K, 256), jnp.float32),
        pltpu.VMEM((K, 256), jnp.float32),
        pltpu.VMEM((K, 256), jnp.float32),
        pltpu.VMEM((K, 256), jnp.float32),
        pltpu.VMEM((K, 256), jnp.float32),
        pltpu.VMEM((K, 256), jnp.float32),
        pltpu.VMEM((K, CONV), jnp.float32),
        pltpu.VMEM_SHARED((N, CONV), jnp.float32),
        pltpu.SemaphoreType.DMA,
        pltpu.SemaphoreType.DMA,
        pltpu.SemaphoreType.DMA,
        pltpu.SemaphoreType.DMA,
        pltpu.SemaphoreType.DMA,
        pltpu.SemaphoreType.DMA,
    ],
)


def _stats_body(part_ref, stats_ref):
    i = pl.program_id(0)
    aggr = part_ref[0] + part_ref[1]
    s = jnp.sum(aggr, axis=0)
    s2 = jnp.sum(aggr * aggr, axis=0)
    st = jnp.stack([s, s2])

    @pl.when(i == 0)
    def _():
        stats_ref[...] = st

    @pl.when(i > 0)
    def _():
        stats_ref[...] += st


_stats = pl.pallas_call(
    _stats_body,
    grid=(NSTEP,),
    in_specs=[pl.BlockSpec((2, BN, 128), lambda i: (0, i, 0))],
    out_specs=pl.BlockSpec((2, 128), lambda i: (0, 0)),
    out_shape=jax.ShapeDtypeStruct((2, 128), jnp.float32),
)


def _apply0_body(part_ref, h_ref, stats_ref, g_ref, be_ref, wl_ref, bl_ref,
                 wd_ref, ws_ref, h1_ref, a_ref, b_ref):
    aggr = part_ref[0] + part_ref[1]
    mean = stats_ref[0:1, :] / N
    var = stats_ref[1:2, :] / N - mean * mean
    inv = lax.rsqrt(var + 1e-5)
    xn = (aggr - mean) * (inv * g_ref[...]) + be_ref[...] + h_ref[...]
    hn = _lr(jnp.dot(xn, wl_ref[...], preferred_element_type=jnp.float32)
             + bl_ref[...])
    h1_ref[...] = hn
    a_ref[...] = jnp.dot(hn, wd_ref[...], preferred_element_type=jnp.float32)
    b_ref[...] = jnp.dot(hn, ws_ref[...], preferred_element_type=jnp.float32)


_apply0 = pl.pallas_call(
    _apply0_body,
    grid=(NSTEP,),
    in_specs=[
        pl.BlockSpec((2, BN, 128), lambda i: (0, i, 0)),
        pl.BlockSpec((BN, 128), lambda i: (i, 0)),
        pl.BlockSpec((2, 128), lambda i: (0, 0)),
        pl.BlockSpec((1, 128), lambda i: (0, 0)),
        pl.BlockSpec((1, 128), lambda i: (0, 0)),
        pl.BlockSpec((128, 128), lambda i: (0, 0)),
        pl.BlockSpec((1, 128), lambda i: (0, 0)),
        pl.BlockSpec((128, 256), lambda i: (0, 0)),
        pl.BlockSpec((128, 256), lambda i: (0, 0)),
    ],
    out_specs=[
        pl.BlockSpec((BN, 128), lambda i: (i, 0)),
        pl.BlockSpec((BN, 256), lambda i: (i, 0)),
        pl.BlockSpec((BN, 256), lambda i: (i, 0)),
    ],
    out_shape=[
        jax.ShapeDtypeStruct((N, 128), jnp.float32),
        jax.ShapeDtypeStruct((N, 256), jnp.float32),
        jax.ShapeDtypeStruct((N, 256), jnp.float32),
    ],
)


def _final_body(part_ref, h_ref, stats_ref, g_ref, be_ref, wl_ref, bl_ref,
                batch_ref, wp_ref, bp_ref, wh_ref, bh_ref, wo_ref, bo_ref,
                out_ref, pooled_ref, cnt_ref):
    i = pl.program_id(0)
    aggr = part_ref[0] + part_ref[1]
    mean = stats_ref[0:1, :] / N
    var = stats_ref[1:2, :] / N - mean * mean
    inv = lax.rsqrt(var + 1e-5)
    xn = (aggr - mean) * (inv * g_ref[...]) + be_ref[...] + h_ref[...]
    h2 = _lr(jnp.dot(xn, wl_ref[...], preferred_element_type=jnp.float32)
             + bl_ref[...])

    batch_blk = batch_ref[0, 0, :]
    gids = lax.broadcasted_iota(jnp.int32, (BN, NG), 1)
    onehot = (batch_blk[:, None] == gids).astype(jnp.float32)
    pooled_c = lax.dot_general(onehot, h2, (((0,), (0,)), ((), ())),
                               preferred_element_type=jnp.float32)
    cnt_c = jnp.broadcast_to(jnp.sum(onehot, axis=0)[:, None], (NG, 128))

    @pl.when(i == 0)
    def _():
        pooled_ref[...] = pooled_c
        cnt_ref[...] = cnt_c

    @pl.when(i > 0)
    def _():
        pooled_ref[...] += pooled_c
        cnt_ref[...] += cnt_c

    @pl.when(i == NSTEP - 1)
    def _():
        pooled = pooled_ref[...] / jnp.maximum(cnt_ref[...], 1.0)
        hid = _lr(jnp.dot(pooled, wp_ref[...], preferred_element_type=jnp.float32)
                  + bp_ref[...])
        hid = _lr(jnp.dot(hid, wh_ref[...], preferred_element_type=jnp.float32)
                  + bh_ref[...])
        out_ref[...] = _lr(jnp.dot(hid, wo_ref[...],
                                   preferred_element_type=jnp.float32)
                           + bo_ref[...])


_final = pl.pallas_call(
    _final_body,
    grid=(NSTEP,),
    in_specs=[
        pl.BlockSpec((2, BN, 128), lambda i: (0, i, 0)),
        pl.BlockSpec((BN, 128), lambda i: (i, 0)),
        pl.BlockSpec((2, 128), lambda i: (0, 0)),
        pl.BlockSpec((1, 128), lambda i: (0, 0)),
        pl.BlockSpec((1, 128), lambda i: (0, 0)),
        pl.BlockSpec((128, 128), lambda i: (0, 0)),
        pl.BlockSpec((1, 128), lambda i: (0, 0)),
        pl.BlockSpec((1, 1, BN), lambda i: (i, 0, 0)),
        pl.BlockSpec((128, HID), lambda i: (0, 0)),
        pl.BlockSpec((1, HID), lambda i: (0, 0)),
        pl.BlockSpec((HID, HID), lambda i: (0, 0)),
        pl.BlockSpec((1, HID), lambda i: (0, 0)),
        pl.BlockSpec((HID, 1), lambda i: (0, 0)),
        pl.BlockSpec((1, 1), lambda i: (0, 0)),
    ],
    out_specs=pl.BlockSpec((NG, 1), lambda i: (0, 0)),
    out_shape=jax.ShapeDtypeStruct((NG, 1), jnp.float32),
    scratch_shapes=[
        pltpu.VMEM((NG, 128), jnp.float32),
        pltpu.VMEM((NG, 128), jnp.float32),
    ],
)


def kernel(x, edge_index, edge_attr, batch,
           W_init, b_init,
           Wf0, bf0, Ws0, bs0, gamma0, beta0, Wl0, bl0,
           Wf1, bf1, Ws1, bs1, gamma1, beta1, Wl1, bl1,
           W_pool, b_pool, W_h1, b_h1, W_out, b_out):
    src = jnp.pad(edge_index[0], (0, E_PAD - E))
    dst = jnp.pad(edge_index[1], (0, E_PAD - E))
    ea_pad = jnp.pad(edge_attr, ((0, E_PAD - E), (0, 0)))
    zrows = jnp.zeros((T0, CONV), jnp.float32)

    Wd0 = jnp.concatenate([Wf0[:CONV], Ws0[:CONV]], axis=1)
    Wsrc0 = jnp.concatenate([Wf0[CONV:2 * CONV], Ws0[CONV:2 * CONV]], axis=1)
    Wd1 = jnp.concatenate([Wf1[:CONV], Ws1[:CONV]], axis=1)
    Wsrc1 = jnp.concatenate([Wf1[CONV:2 * CONV], Ws1[CONV:2 * CONV]], axis=1)
    We = jnp.concatenate([Wf0[2 * CONV:], Ws0[2 * CONV:],
                          Wf1[2 * CONV:], Ws1[2 * CONV:]], axis=1)
    biases = jnp.concatenate([bf0, bs0, bf1, bs1]).reshape(1, 512)

    h0, A0, B0 = _prep(x, W_init, b_init.reshape(1, 128), Wd0, Wsrc0)
    EA0, EA1 = _edgeattr(ea_pad, We, biases)

    part0 = _edge_stage(A0, B0, EA0, dst, src, zrows)
    stats0 = _stats(part0)
    h1, A1, B1 = _apply0(part0, h0, stats0, gamma0.reshape(1, 128),
                         beta0.reshape(1, 128), Wl0, bl0.reshape(1, 128),
                         Wd1, Wsrc1)

    part1 = _edge_stage(A1, B1, EA1, dst, src, zrows)
    stats1 = _stats(part1)

    out = _final(part1, h1, stats1, gamma1.reshape(1, 128),
                 beta1.reshape(1, 128), Wl1, bl1.reshape(1, 128),
                 batch.reshape(NSTEP, 1, BN),
                 W_pool, b_pool.reshape(1, HID),
                 W_h1, b_h1.reshape(1, HID),
                 W_out, b_out.reshape(1, 1))
    return out


# speedup vs baseline: 3.2055x; 3.0828x over previous
import jax
import jax.numpy as jnp
from jax import lax
from jax.experimental import pallas as pl
from jax.experimental.pallas import tpu as pltpu
from jax.experimental.pallas import tpu_sc as plsc

N = 10000
E = 160000
CONV = 128
NG = 64
HID = 512

BN = 1000
NSTEP = N // BN

K = 24
NW = 32
CPW = 210
EPW = K * CPW
E_PAD = EPW * NW
BE = 4032
T0 = 624
TAIL = N - 16 * T0
TAIL_OFF = 16 * T0


def _lr(v):
    return jnp.where(v > 0, v, 0.01 * v)


def _prep_body(x_ref, wi_ref, bi_ref, wd_ref, ws_ref, h_ref, a_ref, b_ref):
    h = _lr(jnp.dot(x_ref[...], wi_ref[...], preferred_element_type=jnp.float32)
            + bi_ref[...])
    h_ref[...] = h
    a_ref[...] = jnp.dot(h, wd_ref[...], preferred_element_type=jnp.float32)
    b_ref[...] = jnp.dot(h, ws_ref[...], preferred_element_type=jnp.float32)


_prep = pl.pallas_call(
    _prep_body,
    grid=(NSTEP,),
    in_specs=[
        pl.BlockSpec((BN, 128), lambda i: (i, 0)),
        pl.BlockSpec((128, 128), lambda i: (0, 0)),
        pl.BlockSpec((1, 128), lambda i: (0, 0)),
        pl.BlockSpec((128, 256), lambda i: (0, 0)),
        pl.BlockSpec((128, 256), lambda i: (0, 0)),
    ],
    out_specs=[
        pl.BlockSpec((BN, 128), lambda i: (i, 0)),
        pl.BlockSpec((BN, 256), lambda i: (i, 0)),
        pl.BlockSpec((BN, 256), lambda i: (i, 0)),
    ],
    out_shape=[
        jax.ShapeDtypeStruct((N, 128), jnp.float32),
        jax.ShapeDtypeStruct((N, 256), jnp.float32),
        jax.ShapeDtypeStruct((N, 256), jnp.float32),
    ],
)


def _edgeattr_body(e_ref, w_ref, b_ref, ea0_ref, ea1_ref):
    i = pl.program_id(0)
    z = jnp.dot(e_ref[...], w_ref[...], preferred_element_type=jnp.float32) + b_ref[...]
    rid = lax.broadcasted_iota(jnp.int32, (BE, 1), 0) + i * BE
    z = jnp.where(rid < E, z, -50.0)
    ea0_ref[...] = z[:, :256]
    ea1_ref[...] = z[:, 256:]


_edgeattr = pl.pallas_call(
    _edgeattr_body,
    grid=(E_PAD // BE,),
    in_specs=[
        pl.BlockSpec((BE, 16), lambda i: (i, 0)),
        pl.BlockSpec((16, 512), lambda i: (0, 0)),
        pl.BlockSpec((1, 512), lambda i: (0, 0)),
    ],
    out_specs=[
        pl.BlockSpec((BE, 256), lambda i: (i, 0)),
        pl.BlockSpec((BE, 256), lambda i: (i, 0)),
    ],
    out_shape=[
        jax.ShapeDtypeStruct((E_PAD, 256), jnp.float32),
        jax.ShapeDtypeStruct((E_PAD, 256), jnp.float32),
    ],
)


def _sc_edge_body(a_hbm, b_hbm, ea_hbm, dst_hbm, src_hbm, z_hbm, out_hbm,
                  dvall, svall, dsc,
                  ab0, bb0, eb0, ab1, bb1, eb1, mb,
                  aggr, sa0, sb0, se0, sa1, sb1, se1):
    cid = lax.axis_index("c")
    sid = lax.axis_index("s")
    wid = cid * 16 + sid

    set0 = (ab0, bb0, eb0, sa0, sb0, se0)
    set1 = (ab1, bb1, eb1, sa1, sb1, se1)

    pltpu.sync_copy(z_hbm, aggr.at[pl.ds(sid * T0, T0)])

    @pl.when(sid == 15)
    def _():
        pltpu.sync_copy(z_hbm.at[pl.ds(0, TAIL)], aggr.at[pl.ds(TAIL_OFF, TAIL)])

    pltpu.sync_copy(dst_hbm.at[pl.ds(wid * EPW, EPW)], dvall)
    pltpu.sync_copy(src_hbm.at[pl.ds(wid * EPW, EPW)], svall)
    plsc.subcore_barrier()

    def issue(i, s):
        ab, bb, eb, sa, sb, se = s
        base = wid * EPW + i * K
        da = pltpu.async_copy(a_hbm.at[dvall.at[pl.ds(i * K, K)]], ab, sa)
        db = pltpu.async_copy(b_hbm.at[svall.at[pl.ds(i * K, K)]], bb, sb)
        de = pltpu.async_copy(ea_hbm.at[pl.ds(base, K)], eb, se)
        return (da, db, de)

    def finish(i, s, descs):
        ab, bb, eb = s[0], s[1], s[2]
        for d in descs:
            d.wait()

        @plsc.parallel_loop(0, K, unroll=2)
        def _(r):
            gfs, gss = [], []
            for j in range(8):
                lo = j * 16
                gfs.append(ab[r, pl.ds(lo, 16)] + bb[r, pl.ds(lo, 16)]
                           + eb[r, pl.ds(lo, 16)])
                gss.append(ab[r, pl.ds(128 + lo, 16)] + bb[r, pl.ds(128 + lo, 16)]
                           + eb[r, pl.ds(128 + lo, 16)])
            sigs = [1.0 / (1.0 + jnp.exp(-g)) for g in gfs]
            us = [jnp.exp(-jnp.abs(g)) for g in gss]
            ts = [u / (2.0 + u) for u in us]
            t2s = [t * t for t in ts]
            qs = [t2 * (1.0 / 7.0) + (1.0 / 5.0) for t2 in t2s]
            qs = [t2 * q + (1.0 / 3.0) for t2, q in zip(t2s, qs)]
            qs = [t2 * q + 1.0 for t2, q in zip(t2s, qs)]
            sps = [jnp.maximum(g, 0.0) + (2.0 * t) * q
                   for g, t, q in zip(gss, ts, qs)]
            for j in range(8):
                mb[r, pl.ds(j * 16, 16)] = sigs[j] * sps[j]
        o = 0
        while o + 16 <= K:
            dsc[pl.ds(o, 16)] = dvall[pl.ds(i * K + o, 16)]
            o += 16
        if o < K:
            dsc[pl.ds(K - 16, 16)] = dvall[pl.ds(i * K + K - 16, 16)]
        pltpu.sync_copy(mb, aggr.at[dsc], add=True)

    @pl.loop(0, CPW // 2)
    def _(t):
        i0 = 2 * t
        i1 = i0 + 1
        d0 = issue(i0, set0)
        d1 = issue(i1, set1)
        finish(i0, set0, d0)
        finish(i1, set1, d1)

    plsc.subcore_barrier()
    pltpu.sync_copy(aggr.at[pl.ds(sid * T0, T0)],
                    out_hbm.at[cid, pl.ds(sid * T0, T0)])

    @pl.when(sid == 15)
    def _():
        pltpu.sync_copy(aggr.at[pl.ds(TAIL_OFF, TAIL)],
                        out_hbm.at[cid, pl.ds(TAIL_OFF, TAIL)])


_edge_stage = pl.kernel(
    _sc_edge_body,
    out_type=jax.ShapeDtypeStruct((2, N, CONV), jnp.float32),
    mesh=plsc.VectorSubcoreMesh(core_axis_name="c", subcore_axis_name="s"),
    scratch_types=[
        pltpu.VMEM((EPW,), jnp.int32),
        pltpu.VMEM((EPW,), jnp.int32),
        pltpu.VMEM((K,), jnp.int32),
        pltpu.VMEM((K, 256), jnp.float32),
        pltpu.VMEM((K, 256), jnp.float32),
        pltpu.VMEM((K, 256), jnp.float32),
        pltpu.VMEM((K, 256), jnp.float32),
        pltpu.VMEM((K, 256), jnp.float32),
        pltpu.VMEM((K, 256), jnp.float32),
        pltpu.VMEM((K, CONV), jnp.float32),
        pltpu.VMEM_SHARED((N, CONV), jnp.float32),
        pltpu.SemaphoreType.DMA,
        pltpu.SemaphoreType.DMA,
        pltpu.SemaphoreType.DMA,
        pltpu.SemaphoreType.DMA,
        pltpu.SemaphoreType.DMA,
        pltpu.SemaphoreType.DMA,
    ],
)


def _stats_body(part_ref, stats_ref):
    i = pl.program_id(0)
    aggr = part_ref[0] + part_ref[1]
    s = jnp.sum(aggr, axis=0)
    s2 = jnp.sum(aggr * aggr, axis=0)
    st = jnp.stack([s, s2])

    @pl.when(i == 0)
    def _():
        stats_ref[...] = st

    @pl.when(i > 0)
    def _():
        stats_ref[...] += st


_stats = pl.pallas_call(
    _stats_body,
    grid=(NSTEP,),
    in_specs=[pl.BlockSpec((2, BN, 128), lambda i: (0, i, 0))],
    out_specs=pl.BlockSpec((2, 128), lambda i: (0, 0)),
    out_shape=jax.ShapeDtypeStruct((2, 128), jnp.float32),
)


def _apply0_body(part_ref, h_ref, stats_ref, g_ref, be_ref, wl_ref, bl_ref,
                 wd_ref, ws_ref, h1_ref, a_ref, b_ref):
    aggr = part_ref[0] + part_ref[1]
    mean = stats_ref[0:1, :] / N
    var = stats_ref[1:2, :] / N - mean * mean
    inv = lax.rsqrt(var + 1e-5)
    xn = (aggr - mean) * (inv * g_ref[...]) + be_ref[...] + h_ref[...]
    hn = _lr(jnp.dot(xn, wl_ref[...], preferred_element_type=jnp.float32)
             + bl_ref[...])
    h1_ref[...] = hn
    a_ref[...] = jnp.dot(hn, wd_ref[...], preferred_element_type=jnp.float32)
    b_ref[...] = jnp.dot(hn, ws_ref[...], preferred_element_type=jnp.float32)


_apply0 = pl.pallas_call(
    _apply0_body,
    grid=(NSTEP,),
    in_specs=[
        pl.BlockSpec((2, BN, 128), lambda i: (0, i, 0)),
        pl.BlockSpec((BN, 128), lambda i: (i, 0)),
        pl.BlockSpec((2, 128), lambda i: (0, 0)),
        pl.BlockSpec((1, 128), lambda i: (0, 0)),
        pl.BlockSpec((1, 128), lambda i: (0, 0)),
        pl.BlockSpec((128, 128), lambda i: (0, 0)),
        pl.BlockSpec((1, 128), lambda i: (0, 0)),
        pl.BlockSpec((128, 256), lambda i: (0, 0)),
        pl.BlockSpec((128, 256), lambda i: (0, 0)),
    ],
    out_specs=[
        pl.BlockSpec((BN, 128), lambda i: (i, 0)),
        pl.BlockSpec((BN, 256), lambda i: (i, 0)),
        pl.BlockSpec((BN, 256), lambda i: (i, 0)),
    ],
    out_shape=[
        jax.ShapeDtypeStruct((N, 128), jnp.float32),
        jax.ShapeDtypeStruct((N, 256), jnp.float32),
        jax.ShapeDtypeStruct((N, 256), jnp.float32),
    ],
)


def _final_body(part_ref, h_ref, stats_ref, g_ref, be_ref, wl_ref, bl_ref,
                batch_ref, wp_ref, bp_ref, wh_ref, bh_ref, wo_ref, bo_ref,
                out_ref, pooled_ref, cnt_ref):
    i = pl.program_id(0)
    aggr = part_ref[0] + part_ref[1]
    mean = stats_ref[0:1, :] / N
    var = stats_ref[1:2, :] / N - mean * mean
    inv = lax.rsqrt(var + 1e-5)
    xn = (aggr - mean) * (inv * g_ref[...]) + be_ref[...] + h_ref[...]
    h2 = _lr(jnp.dot(xn, wl_ref[...], preferred_element_type=jnp.float32)
             + bl_ref[...])

    batch_blk = batch_ref[0, 0, :]
    gids = lax.broadcasted_iota(jnp.int32, (BN, NG), 1)
    onehot = (batch_blk[:, None] == gids).astype(jnp.float32)
    pooled_c = lax.dot_general(onehot, h2, (((0,), (0,)), ((), ())),
                               preferred_element_type=jnp.float32)
    cnt_c = jnp.broadcast_to(jnp.sum(onehot, axis=0)[:, None], (NG, 128))

    @pl.when(i == 0)
    def _():
        pooled_ref[...] = pooled_c
        cnt_ref[...] = cnt_c

    @pl.when(i > 0)
    def _():
        pooled_ref[...] += pooled_c
        cnt_ref[...] += cnt_c

    @pl.when(i == NSTEP - 1)
    def _():
        pooled = pooled_ref[...] / jnp.maximum(cnt_ref[...], 1.0)
        hid = _lr(jnp.dot(pooled, wp_ref[...], preferred_element_type=jnp.float32)
                  + bp_ref[...])
        hid = _lr(jnp.dot(hid, wh_ref[...], preferred_element_type=jnp.float32)
                  + bh_ref[...])
        out_ref[...] = _lr(jnp.dot(hid, wo_ref[...],
                                   preferred_element_type=jnp.float32)
                           + bo_ref[...])


_final = pl.pallas_call(
    _final_body,
    grid=(NSTEP,),
    in_specs=[
        pl.BlockSpec((2, BN, 128), lambda i: (0, i, 0)),
        pl.BlockSpec((BN, 128), lambda i: (i, 0)),
        pl.BlockSpec((2, 128), lambda i: (0, 0)),
        pl.BlockSpec((1, 128), lambda i: (0, 0)),
        pl.BlockSpec((1, 128), lambda i: (0, 0)),
        pl.BlockSpec((128, 128), lambda i: (0, 0)),
        pl.BlockSpec((1, 128), lambda i: (0, 0)),
        pl.BlockSpec((1, 1, BN), lambda i: (i, 0, 0)),
        pl.BlockSpec((128, HID), lambda i: (0, 0)),
        pl.BlockSpec((1, HID), lambda i: (0, 0)),
        pl.BlockSpec((HID, HID), lambda i: (0, 0)),
        pl.BlockSpec((1, HID), lambda i: (0, 0)),
        pl.BlockSpec((HID, 1), lambda i: (0, 0)),
        pl.BlockSpec((1, 1), lambda i: (0, 0)),
    ],
    out_specs=pl.BlockSpec((NG, 1), lambda i: (0, 0)),
    out_shape=jax.ShapeDtypeStruct((NG, 1), jnp.float32),
    scratch_shapes=[
        pltpu.VMEM((NG, 128), jnp.float32),
        pltpu.VMEM((NG, 128), jnp.float32),
    ],
)


def kernel(x, edge_index, edge_attr, batch,
           W_init, b_init,
           Wf0, bf0, Ws0, bs0, gamma0, beta0, Wl0, bl0,
           Wf1, bf1, Ws1, bs1, gamma1, beta1, Wl1, bl1,
           W_pool, b_pool, W_h1, b_h1, W_out, b_out):
    src = jnp.pad(edge_index[0], (0, E_PAD - E))
    dst = jnp.pad(edge_index[1], (0, E_PAD - E))
    ea_pad = jnp.pad(edge_attr, ((0, E_PAD - E), (0, 0)))
    zrows = jnp.zeros((T0, CONV), jnp.float32)

    Wd0 = jnp.concatenate([Wf0[:CONV], Ws0[:CONV]], axis=1)
    Wsrc0 = jnp.concatenate([Wf0[CONV:2 * CONV], Ws0[CONV:2 * CONV]], axis=1)
    Wd1 = jnp.concatenate([Wf1[:CONV], Ws1[:CONV]], axis=1)
    Wsrc1 = jnp.concatenate([Wf1[CONV:2 * CONV], Ws1[CONV:2 * CONV]], axis=1)
    We = jnp.concatenate([Wf0[2 * CONV:], Ws0[2 * CONV:],
                          Wf1[2 * CONV:], Ws1[2 * CONV:]], axis=1)
    biases = jnp.concatenate([bf0, bs0, bf1, bs1]).reshape(1, 512)

    h0, A0, B0 = _prep(x, W_init, b_init.reshape(1, 128), Wd0, Wsrc0)
    EA0, EA1 = _edgeattr(ea_pad, We, biases)

    part0 = _edge_stage(A0, B0, EA0, dst, src, zrows)
    stats0 = _stats(part0)
    h1, A1, B1 = _apply0(part0, h0, stats0, gamma0.reshape(1, 128),
                         beta0.reshape(1, 128), Wl0, bl0.reshape(1, 128),
                         Wd1, Wsrc1)

    part1 = _edge_stage(A1, B1, EA1, dst, src, zrows)
    stats1 = _stats(part1)

    out = _final(part1, h1, stats1, gamma1.reshape(1, 128),
                 beta1.reshape(1, 128), Wl1, bl1.reshape(1, 128),
                 batch.reshape(NSTEP, 1, BN),
                 W_pool, b_pool.reshape(1, HID),
                 W_h1, b_h1.reshape(1, HID),
                 W_out, b_out.reshape(1, 1))
    return out
